# Initial kernel scaffold; baseline (speedup 1.0000x reference)
import jax
import jax.numpy as jnp
from jax.experimental import pallas as pl


def kernel(x_nchw, weight):
    raise NotImplementedError("write your pallas kernel here")


# speedup vs baseline: 1.1337x; 1.1337x over previous
import jax
import jax.numpy as jnp
from jax.experimental import pallas as pl
from jax.experimental.pallas import tpu as pltpu


def _wsum_kernel(x_ref, w_ref, o_ref):
    x = x_ref[...]
    w = w_ref[...]
    o_ref[...] = jnp.sum(x * w[None, :, :], axis=1)


def kernel(x_nchw, weight):
    N, C_in, H, W = x_nchw.shape
    C_out = weight.shape[0]
    HW = H * W
    w_col = weight.reshape(C_out * C_in, 1).astype(jnp.float32)

    Bn = 1
    for cand in (8, 4, 2):
        if N % cand == 0 and N // cand >= 2:
            Bn = cand
            break

    x = x_nchw.reshape(N, C_in, HW)
    in_bytes = Bn * C_in * HW * x.dtype.itemsize
    out_bytes = Bn * HW * x.dtype.itemsize
    vmem = int(min(2 * (2 * (in_bytes + out_bytes)) + 4096 * 4, 100 << 20))

    out = pl.pallas_call(
        _wsum_kernel,
        out_shape=jax.ShapeDtypeStruct((N, HW), x_nchw.dtype),
        grid=(N // Bn,),
        in_specs=[
            pl.BlockSpec((Bn, C_in, HW), lambda i: (i, 0, 0)),
            pl.BlockSpec((C_in, 1), lambda i: (0, 0)),
        ],
        out_specs=pl.BlockSpec((Bn, HW), lambda i: (i, 0)),
        compiler_params=pltpu.CompilerParams(
            dimension_semantics=("parallel",),
            vmem_limit_bytes=vmem,
        ),
    )(x, w_col)
    return out.reshape(N, C_out, H, W)
